# baseline (speedup 1.0000x reference)
import jax
import jax.numpy as jnp
from jax import lax
from jax.experimental import pallas as pl
from jax.experimental.pallas import tpu as pltpu
from jax.experimental.pallas import tpu_sc as plsc

_NUM_CORES = 2
_NUM_SUBCORES = 16
_CHUNK = 16
_NBUF = 3
_CHUNKS_C0 = 19
_CHUNKS_C1 = 13
_ROWS_C0 = _NUM_SUBCORES * _CHUNK * _CHUNKS_C0


def _pipeline(src_hbm, dst_hbm, bufs, isems, osems, base, nchunks):
    in_c = [None] * _NBUF
    out_c = [None] * _NBUF
    for i in range(nchunks):
        b = i % _NBUF
        if out_c[b] is not None:
            out_c[b].wait()
        lo = base + i * _CHUNK
        in_c[b] = pltpu.async_copy(src_hbm.at[pl.ds(lo, _CHUNK)], bufs[b], isems[b])
        if i > 0:
            pb = (i - 1) % _NBUF
            in_c[pb].wait()
            plo = base + (i - 1) * _CHUNK
            out_c[pb] = pltpu.async_copy(bufs[pb], dst_hbm.at[pl.ds(plo, _CHUNK)], osems[pb])
    lb = (nchunks - 1) % _NBUF
    in_c[lb].wait()
    llo = base + (nchunks - 1) * _CHUNK
    out_c[lb] = pltpu.async_copy(bufs[lb], dst_hbm.at[pl.ds(llo, _CHUNK)], osems[lb])
    for b in range(_NBUF):
        if out_c[b] is not None:
            out_c[b].wait()


def _sc_body(emb_hbm, out_hbm, *scratch):
    bufs = list(scratch[:_NBUF])
    isems = list(scratch[_NBUF : 2 * _NBUF])
    osems = list(scratch[2 * _NBUF : 3 * _NBUF])
    cid = lax.axis_index("c")
    sid = lax.axis_index("s")

    @pl.when(cid == 0)
    def _():
        base = sid * (_CHUNK * _CHUNKS_C0)
        _pipeline(emb_hbm, out_hbm, bufs, isems, osems, base, _CHUNKS_C0)

    @pl.when(cid == 1)
    def _():
        base = _ROWS_C0 + sid * (_CHUNK * _CHUNKS_C1)
        _pipeline(emb_hbm, out_hbm, bufs, isems, osems, base, _CHUNKS_C1)


def kernel(x, emb):
    seq_len = x.shape[1]
    d = emb.shape[1]
    mesh = plsc.VectorSubcoreMesh(core_axis_name="c", subcore_axis_name="s")
    out = pl.kernel(
        _sc_body,
        out_type=jax.ShapeDtypeStruct((seq_len, d), emb.dtype),
        mesh=mesh,
        scratch_types=(
            [pltpu.VMEM((_CHUNK, d), jnp.float32)] * _NBUF
            + [pltpu.SemaphoreType.DMA] * (2 * _NBUF)
        ),
    )(emb)
    return out[None]


# speedup vs baseline: 1.0185x; 1.0185x over previous
import jax
import jax.numpy as jnp
from jax import lax
from jax.experimental import pallas as pl
from jax.experimental.pallas import tpu as pltpu
from jax.experimental.pallas import tpu_sc as plsc

_NUM_CORES = 2
_NUM_SUBCORES = 16
_CHUNK = 16
_NBUF = 3
_CHUNKS_C0 = 13
_CHUNKS_C1 = 19
_ROWS_C0 = _NUM_SUBCORES * _CHUNK * _CHUNKS_C0


def _pipeline(src_hbm, dst_hbm, bufs, isems, osems, base, nchunks):
    in_c = [None] * _NBUF
    out_c = [None] * _NBUF
    for i in range(nchunks):
        b = i % _NBUF
        if out_c[b] is not None:
            out_c[b].wait()
        lo = base + i * _CHUNK
        in_c[b] = pltpu.async_copy(src_hbm.at[pl.ds(lo, _CHUNK)], bufs[b], isems[b])
        if i > 0:
            pb = (i - 1) % _NBUF
            in_c[pb].wait()
            plo = base + (i - 1) * _CHUNK
            out_c[pb] = pltpu.async_copy(bufs[pb], dst_hbm.at[pl.ds(plo, _CHUNK)], osems[pb])
    lb = (nchunks - 1) % _NBUF
    in_c[lb].wait()
    llo = base + (nchunks - 1) * _CHUNK
    out_c[lb] = pltpu.async_copy(bufs[lb], dst_hbm.at[pl.ds(llo, _CHUNK)], osems[lb])
    for b in range(_NBUF):
        if out_c[b] is not None:
            out_c[b].wait()


def _sc_body(emb_hbm, out_hbm, *scratch):
    bufs = list(scratch[:_NBUF])
    isems = list(scratch[_NBUF : 2 * _NBUF])
    osems = list(scratch[2 * _NBUF : 3 * _NBUF])
    cid = lax.axis_index("c")
    sid = lax.axis_index("s")

    @pl.when(cid == 0)
    def _():
        base = sid * (_CHUNK * _CHUNKS_C0)
        _pipeline(emb_hbm, out_hbm, bufs, isems, osems, base, _CHUNKS_C0)

    @pl.when(cid == 1)
    def _():
        base = _ROWS_C0 + sid * (_CHUNK * _CHUNKS_C1)
        _pipeline(emb_hbm, out_hbm, bufs, isems, osems, base, _CHUNKS_C1)


def kernel(x, emb):
    seq_len = x.shape[1]
    d = emb.shape[1]
    mesh = plsc.VectorSubcoreMesh(core_axis_name="c", subcore_axis_name="s")
    out = pl.kernel(
        _sc_body,
        out_type=jax.ShapeDtypeStruct((seq_len, d), emb.dtype),
        mesh=mesh,
        scratch_types=(
            [pltpu.VMEM((_CHUNK, d), jnp.float32)] * _NBUF
            + [pltpu.SemaphoreType.DMA] * (2 * _NBUF)
        ),
    )(emb)
    return out[None]


# speedup vs baseline: 1.0774x; 1.0579x over previous
import jax
import jax.numpy as jnp
from jax import lax
from jax.experimental import pallas as pl
from jax.experimental.pallas import tpu as pltpu
from jax.experimental.pallas import tpu_sc as plsc

_NUM_CORES = 2
_CHUNK = 256
_NBUF = 3


def _sc_body(emb_hbm, out_hbm, *scratch):
    bufs = list(scratch[:_NBUF])
    isems = list(scratch[_NBUF : 2 * _NBUF])
    osems = list(scratch[2 * _NBUF : 3 * _NBUF])
    cid = lax.axis_index("c")
    rows = emb_hbm.shape[0] // _NUM_CORES
    base = cid * rows
    nchunks = rows // _CHUNK
    in_c = [None] * _NBUF
    out_c = [None] * _NBUF
    for i in range(nchunks):
        b = i % _NBUF
        if out_c[b] is not None:
            out_c[b].wait()
        lo = base + i * _CHUNK
        in_c[b] = pltpu.async_copy(emb_hbm.at[pl.ds(lo, _CHUNK)], bufs[b], isems[b])
        if i > 0:
            pb = (i - 1) % _NBUF
            in_c[pb].wait()
            plo = base + (i - 1) * _CHUNK
            out_c[pb] = pltpu.async_copy(bufs[pb], out_hbm.at[pl.ds(plo, _CHUNK)], osems[pb])
    lb = (nchunks - 1) % _NBUF
    in_c[lb].wait()
    llo = base + (nchunks - 1) * _CHUNK
    out_c[lb] = pltpu.async_copy(bufs[lb], out_hbm.at[pl.ds(llo, _CHUNK)], osems[lb])
    for b in range(_NBUF):
        if out_c[b] is not None:
            out_c[b].wait()


def kernel(x, emb):
    seq_len = x.shape[1]
    d = emb.shape[1]
    mesh = plsc.ScalarSubcoreMesh(axis_name="c", num_cores=_NUM_CORES)
    out = pl.kernel(
        _sc_body,
        out_type=jax.ShapeDtypeStruct((seq_len, d), emb.dtype),
        mesh=mesh,
        scratch_types=(
            [pltpu.VMEM_SHARED((_CHUNK, d), jnp.float32)] * _NBUF
            + [pltpu.SemaphoreType.DMA] * (2 * _NBUF)
        ),
    )(emb)
    return out[None]
